# Initial kernel scaffold; baseline (speedup 1.0000x reference)
import functools

import jax
import jax.numpy as jnp
from jax import lax
from jax.experimental import pallas as pl
from jax.experimental.pallas import tpu as pltpu
from jax.experimental.pallas import tpu_sc as plsc

N_NODES = 10000
D_FEAT = 128
D_HALF = 64
N_EDGES = 320000
BLK = 128
NBLK = N_EDGES // BLK
NSUB = 16
ROWS_PER_TILE = N_NODES // NSUB
ZCHUNK = 125

_mesh = plsc.VectorSubcoreMesh(core_axis_name="c", subcore_axis_name="s")


@functools.partial(
    pl.kernel,
    mesh=_mesh,
    out_type=jax.ShapeDtypeStruct((2 * N_NODES, D_HALF), jnp.float32),
    scratch_types=[
        pltpu.VMEM((BLK,), jnp.int32),
        pltpu.VMEM((BLK,), jnp.int32),
        pltpu.VMEM((BLK, D_HALF), jnp.float32),
        pltpu.VMEM((ZCHUNK, D_HALF), jnp.float32),
        pltpu.VMEM_SHARED((N_NODES, D_HALF), jnp.float32),
        pltpu.SemaphoreType.DMA,
    ],
)
def _sc_agg(xf_hbm, src_hbm, dst_hbm, out_hbm,
            idx_v, dst_v, rows_v, zrows_v, acc_sh, sem):
    c = lax.axis_index("c")
    s = lax.axis_index("s")

    def zrow(i, carry):
        for j in range(D_HALF // 16):
            zrows_v[i, pl.ds(j * 16, 16)] = jnp.zeros((16,), jnp.float32)
        return carry

    lax.fori_loop(0, ZCHUNK, zrow, 0)
    for k in range(ROWS_PER_TILE // ZCHUNK):
        pltpu.sync_copy(
            zrows_v, acc_sh.at[pl.ds(s * ROWS_PER_TILE + k * ZCHUNK, ZCHUNK)])
    plsc.subcore_barrier()

    base_blocks = NBLK // NSUB
    extra = NBLK - base_blocks * NSUB
    nb = jnp.where(s < extra, base_blocks + 1, base_blocks)
    start = s * base_blocks + jnp.minimum(s, extra)
    coff = c * N_NODES

    def blk(i, carry):
        base = (start + i) * BLK
        pltpu.sync_copy(src_hbm.at[pl.ds(base, BLK)], idx_v)
        pltpu.sync_copy(dst_hbm.at[pl.ds(base, BLK)], dst_v)
        for j in range(BLK // 16):
            idx_v[pl.ds(j * 16, 16)] = idx_v[pl.ds(j * 16, 16)] + coff
        pltpu.async_copy(xf_hbm.at[idx_v], rows_v, sem).wait()
        pltpu.sync_copy(rows_v, acc_sh.at[dst_v], add=True)
        return carry

    lax.fori_loop(0, nb, blk, 0)

    plsc.subcore_barrier()
    pltpu.sync_copy(
        acc_sh.at[pl.ds(s * ROWS_PER_TILE, ROWS_PER_TILE)],
        out_hbm.at[pl.ds(coff + s * ROWS_PER_TILE, ROWS_PER_TILE)])


def kernel(x, edge_index):
    src = edge_index[0].astype(jnp.int32)
    dst = edge_index[1].astype(jnp.int32)
    xf = jnp.concatenate([x[:, :D_HALF], x[:, D_HALF:]], axis=0)
    out = _sc_agg(xf, src, dst)
    return jnp.concatenate([out[:N_NODES], out[N_NODES:]], axis=1)


# speedup vs baseline: 6.4908x; 6.4908x over previous
import functools

import jax
import jax.numpy as jnp
from jax import lax
from jax.experimental import pallas as pl
from jax.experimental.pallas import tpu as pltpu
from jax.experimental.pallas import tpu_sc as plsc

N_NODES = 10000
N_PAD = 10240
D_FEAT = 128
N_EDGES = 320000
BLK = 128
NBLK = N_EDGES // BLK
NCORE = 2
NSUB = 16
BLK_PER_CORE = NBLK // NCORE
ROWS_PER_TILE = N_PAD // NSUB
ZCHUNK = 128

_mesh = plsc.VectorSubcoreMesh(core_axis_name="c", subcore_axis_name="s")


@functools.partial(
    pl.kernel,
    mesh=_mesh,
    out_type=jax.ShapeDtypeStruct((NCORE, N_PAD, D_FEAT), jnp.float32),
    scratch_types=[
        pltpu.VMEM((BLK,), jnp.int32),
        pltpu.VMEM((BLK,), jnp.int32),
        pltpu.VMEM((BLK, D_FEAT), jnp.float32),
        pltpu.VMEM((ZCHUNK, D_FEAT), jnp.float32),
        pltpu.VMEM_SHARED((N_PAD, D_FEAT), jnp.float32),
        pltpu.SemaphoreType.DMA,
    ],
)
def _sc_agg(x_hbm, src_hbm, dst_hbm, out_hbm,
            idx_v, dst_v, rows_v, zrows_v, acc_sh, sem):
    c = lax.axis_index("c")
    s = lax.axis_index("s")

    def zrow(i, carry):
        for j in range(D_FEAT // 16):
            zrows_v[i, pl.ds(j * 16, 16)] = jnp.zeros((16,), jnp.float32)
        return carry

    lax.fori_loop(0, ZCHUNK, zrow, 0)
    for k in range(ROWS_PER_TILE // ZCHUNK):
        pltpu.sync_copy(
            zrows_v, acc_sh.at[pl.ds(s * ROWS_PER_TILE + k * ZCHUNK, ZCHUNK)])
    plsc.subcore_barrier()

    base_blocks = BLK_PER_CORE // NSUB
    extra = BLK_PER_CORE - base_blocks * NSUB
    nb = jnp.where(s < extra, base_blocks + 1, base_blocks)
    start = c * BLK_PER_CORE + s * base_blocks + jnp.minimum(s, extra)

    def blk(i, carry):
        base = (start + i) * BLK
        pltpu.sync_copy(src_hbm.at[pl.ds(base, BLK)], idx_v)
        pltpu.sync_copy(dst_hbm.at[pl.ds(base, BLK)], dst_v)
        pltpu.async_copy(x_hbm.at[idx_v], rows_v, sem).wait()
        pltpu.sync_copy(rows_v, acc_sh.at[dst_v], add=True)
        return carry

    lax.fori_loop(0, nb, blk, 0)

    plsc.subcore_barrier()
    pltpu.sync_copy(
        acc_sh.at[pl.ds(s * ROWS_PER_TILE, ROWS_PER_TILE)],
        out_hbm.at[c, pl.ds(s * ROWS_PER_TILE, ROWS_PER_TILE)])


def _add_body(a_ref, b_ref, o_ref):
    o_ref[...] = a_ref[...] + b_ref[...]


_tc_add = pl.pallas_call(
    _add_body,
    out_shape=jax.ShapeDtypeStruct((N_PAD, D_FEAT), jnp.float32),
    grid=(10,),
    in_specs=[
        pl.BlockSpec((N_PAD // 10, D_FEAT), lambda i: (i, 0)),
        pl.BlockSpec((N_PAD // 10, D_FEAT), lambda i: (i, 0)),
    ],
    out_specs=pl.BlockSpec((N_PAD // 10, D_FEAT), lambda i: (i, 0)),
)


def kernel(x, edge_index):
    src = edge_index[0].astype(jnp.int32)
    dst = edge_index[1].astype(jnp.int32)
    xp = jnp.pad(x, ((0, N_PAD - N_NODES), (0, 0)))
    parts = _sc_agg(xp, src, dst)
    out = _tc_add(parts[0], parts[1])
    return out[:N_NODES]
